# baseline (speedup 1.0000x reference)
import functools

import jax
import jax.numpy as jnp
from jax import lax
from jax.experimental import pallas as pl
from jax.experimental.pallas import tpu as pltpu
from jax.experimental.pallas import tpu_sc as plsc

N = 10000
E = 320000
D = 128
H = 8
F = 16
HF = H * F
ALPHA = 0.2
AW = HF + 16
BW = 16
LANES = 16

CHUNK = 80
NTILES = 32
EDGES_PER_TILE = E // NTILES
NK = EDGES_PER_TILE // CHUNK
ROWS_PER_TILE = N // 16


def _prep_body(feat_ref, ma_ref, mb_ref, a_ref, b_ref):
    f = feat_ref[...]
    dn = (((1,), (0,)), ((), ()))
    a_ref[...] = lax.dot_general(f, ma_ref[...], dn,
                                 preferred_element_type=jnp.float32)
    b_ref[...] = lax.dot_general(f, mb_ref[...], dn,
                                 preferred_element_type=jnp.float32)


def _prep(feat, ma, mb):
    return pl.pallas_call(
        _prep_body,
        out_shape=(
            jax.ShapeDtypeStruct((N, AW), jnp.float32),
            jax.ShapeDtypeStruct((N, BW), jnp.float32),
        ),
    )(feat, ma, mb)


def _bcast_lane(vec, lane):
    idx = jnp.full((LANES, 1), lane, dtype=jnp.int32)
    dn = lax.GatherDimensionNumbers(
        offset_dims=(), collapsed_slice_dims=(0,), start_index_map=(0,))
    return lax.gather(vec, idx, dn, (1,),
                      mode=lax.GatherScatterMode.PROMISE_IN_BOUNDS)


def _sc_edge_body(a_hbm, b_hbm, ei_hbm, z_hbm, out_hbm,
                  src0, src1, dst0, dst1, dsts0, dsts1,
                  ar0, ar1, br0, br1, acc,
                  g0, g1, gi0, gi1, sc0, sc1):
    c = lax.axis_index("c")
    s = lax.axis_index("s")
    wid = s * 2 + c
    base = wid * EDGES_PER_TILE

    src = (src0, src1)
    dst = (dst0, dst1)
    dsts = (dsts0, dsts1)
    ar = (ar0, ar1)
    br = (br0, br1)
    g = (g0, g1)
    gi = (gi0, gi1)
    sc = (sc0, sc1)

    r0 = s * ROWS_PER_TILE
    pltpu.sync_copy(z_hbm, acc.at[pl.ds(r0, ROWS_PER_TILE)])
    plsc.subcore_barrier()

    def issue_idx(k, p):
        off = base + k * CHUNK
        pltpu.async_copy(ei_hbm.at[0, pl.ds(off, CHUNK)], src[p], gi[p])
        pltpu.async_copy(ei_hbm.at[1, pl.ds(off, CHUNK)], dst[p], gi[p])

    def wait_idx(p):
        pltpu.make_async_copy(ei_hbm.at[0, pl.ds(0, CHUNK)], src[p], gi[p]).wait()
        pltpu.make_async_copy(ei_hbm.at[1, pl.ds(0, CHUNK)], dst[p], gi[p]).wait()

    def issue_gather(p):
        pltpu.async_copy(a_hbm.at[src[p]], ar[p], g[p])
        pltpu.async_copy(b_hbm.at[dst[p]], br[p], g[p])

    def wait_gather(p):
        pltpu.make_async_copy(a_hbm.at[src[p]], ar[p], g[p]).wait()
        pltpu.make_async_copy(b_hbm.at[dst[p]], br[p], g[p]).wait()

    def issue_scatter(p):
        pltpu.async_copy(ar[p], acc.at[dsts[p]], sc[p], add=True)

    def wait_scatter(p):
        pltpu.make_async_copy(ar[p], acc.at[dsts[p]], sc[p]).wait()

    def one_iter(k, p, pref_gather, pref_idx, wait_sc):
        q = 1 - p
        wait_gather(p)
        for j in range(0, CHUNK, LANES):
            dsts[p][pl.ds(j, LANES)] = dst[p][pl.ds(j, LANES)]
        if pref_gather:
            wait_idx(q)
            if wait_sc:
                wait_scatter(q)
            issue_gather(q)
        elif wait_sc:
            wait_scatter(q)
        if pref_idx:
            issue_idx(k + 2, p)

        @pl.loop(0, CHUNK)
        def _(i):
            elv = ar[p][i, pl.ds(HF, LANES)]
            erv = br[p][i, :]
            sv = elv + erv
            wv = jnp.exp(jnp.maximum(sv, sv * ALPHA))
            ar[p][i, pl.ds(HF, LANES)] = wv
            for h in range(H):
                wb = _bcast_lane(wv, h)
                ftv = ar[p][i, pl.ds(h * F, LANES)]
                ar[p][i, pl.ds(h * F, LANES)] = ftv * wb

        issue_scatter(p)

    issue_idx(0, 0)
    issue_idx(1, 1)
    wait_idx(0)
    issue_gather(0)
    one_iter(0, 0, True, True, False)
    one_iter(1, 1, True, True, True)

    @pl.loop(1, (NK - 3) // 2)
    def _(kk):
        k = kk * 2
        one_iter(k, 0, True, True, True)
        one_iter(k + 1, 1, True, True, True)

    one_iter(NK - 3, 0, True, True, True)
    one_iter(NK - 2, 1, True, False, True)
    one_iter(NK - 1, 0, False, False, True)
    wait_scatter(0)

    plsc.subcore_barrier()
    pltpu.sync_copy(acc.at[pl.ds(r0, ROWS_PER_TILE)],
                    out_hbm.at[c, pl.ds(r0, ROWS_PER_TILE)])


def _sc_edge(a, b, edge_index, zeros):
    mesh = plsc.VectorSubcoreMesh(core_axis_name="c", subcore_axis_name="s")
    kern = pl.kernel(
        _sc_edge_body,
        out_type=jax.ShapeDtypeStruct((2, N, AW), jnp.float32),
        mesh=mesh,
        scratch_types=[
            pltpu.VMEM((CHUNK,), jnp.int32),
            pltpu.VMEM((CHUNK,), jnp.int32),
            pltpu.VMEM((CHUNK,), jnp.int32),
            pltpu.VMEM((CHUNK,), jnp.int32),
            pltpu.VMEM((CHUNK,), jnp.int32),
            pltpu.VMEM((CHUNK,), jnp.int32),
            pltpu.VMEM((CHUNK, AW), jnp.float32),
            pltpu.VMEM((CHUNK, AW), jnp.float32),
            pltpu.VMEM((CHUNK, BW), jnp.float32),
            pltpu.VMEM((CHUNK, BW), jnp.float32),
            pltpu.VMEM_SHARED((N, AW), jnp.float32),
            pltpu.SemaphoreType.DMA,
            pltpu.SemaphoreType.DMA,
            pltpu.SemaphoreType.DMA,
            pltpu.SemaphoreType.DMA,
            pltpu.SemaphoreType.DMA,
            pltpu.SemaphoreType.DMA,
        ],
        compiler_params=pltpu.CompilerParams(use_tc_tiling_on_sc=False),
    )
    return kern(a, b, edge_index, zeros)


def _div_body(acc_ref, r_ref, o_ref):
    sacc = acc_ref[0] + acc_ref[1]
    den = sacc[:, HF:HF + H]
    dn = (((1,), (0,)), ((), ()))
    denb = lax.dot_general(den, r_ref[...], dn,
                           preferred_element_type=jnp.float32)
    num = sacc[:, :HF]
    o_ref[...] = jnp.where(denb > 0.0, num / denb, 0.0)


def _div(acc, r):
    return pl.pallas_call(
        _div_body,
        out_shape=jax.ShapeDtypeStruct((N, HF), jnp.float32),
    )(acc, r)


def kernel(feat, edge_index, W, attn_l, attn_r):
    f32 = jnp.float32
    wt = W.T.astype(f32)
    eye = jnp.eye(H, dtype=f32)
    wl = (attn_l[0][:, :, None] * eye[:, None, :]).reshape(HF, H)
    wr = (attn_r[0][:, :, None] * eye[:, None, :]).reshape(HF, H)
    wl = jnp.pad(wl, ((0, 0), (0, BW - H)))
    wr = jnp.pad(wr, ((0, 0), (0, BW - H)))
    ma = jnp.concatenate([wt, wt @ wl], axis=1)
    mb = wt @ wr

    a, b = _prep(feat, ma, mb)
    zeros = jnp.zeros((ROWS_PER_TILE, AW), f32)
    acc = _sc_edge(a, b, edge_index, zeros)

    r = jnp.repeat(eye, F, axis=1)
    out = _div(acc, r)
    return out.reshape(N, H, F)


# speedup vs baseline: 1.1721x; 1.1721x over previous
import functools

import jax
import jax.numpy as jnp
from jax import lax
from jax.experimental import pallas as pl
from jax.experimental.pallas import tpu as pltpu
from jax.experimental.pallas import tpu_sc as plsc

N = 10000
E = 320000
D = 128
H = 8
F = 16
HF = H * F
ALPHA = 0.2
AW = HF + 16
BW = 16
LANES = 16

CHUNK = 40
NTILES = 32
EDGES_PER_TILE = E // NTILES
NK = EDGES_PER_TILE // CHUNK
ROWS_PER_TILE = N // 16


def _prep_body(feat_ref, ma_ref, mb_ref, a_ref, b_ref):
    f = feat_ref[...]
    dn = (((1,), (0,)), ((), ()))
    a_ref[...] = lax.dot_general(f, ma_ref[...], dn,
                                 preferred_element_type=jnp.float32)
    b_ref[...] = lax.dot_general(f, mb_ref[...], dn,
                                 preferred_element_type=jnp.float32)


def _prep(feat, ma, mb):
    return pl.pallas_call(
        _prep_body,
        out_shape=(
            jax.ShapeDtypeStruct((N, AW), jnp.float32),
            jax.ShapeDtypeStruct((N, BW), jnp.float32),
        ),
    )(feat, ma, mb)


def _bcast_lane(vec, lane):
    idx = jnp.full((LANES, 1), lane, dtype=jnp.int32)
    dn = lax.GatherDimensionNumbers(
        offset_dims=(), collapsed_slice_dims=(0,), start_index_map=(0,))
    return lax.gather(vec, idx, dn, (1,),
                      mode=lax.GatherScatterMode.PROMISE_IN_BOUNDS)


def _sc_edge_body(a_hbm, b_hbm, ei_hbm, z_hbm, out_hbm,
                  src0, src1, src2, dst0, dst1, dst2, dsts0, dsts1,
                  ar0, ar1, ar2, br0, br1, br2, mr0, mr1, acc,
                  g0, g1, g2, gi0, gi1, gi2, sc0, sc1):
    c = lax.axis_index("c")
    s = lax.axis_index("s")
    wid = s * 2 + c
    base = wid * EDGES_PER_TILE

    src = (src0, src1, src2)
    dst = (dst0, dst1, dst2)
    dsts = (dsts0, dsts1)
    ar = (ar0, ar1, ar2)
    br = (br0, br1, br2)
    mr = (mr0, mr1)
    g = (g0, g1, g2)
    gi = (gi0, gi1, gi2)
    sc = (sc0, sc1)

    r0 = s * ROWS_PER_TILE
    pltpu.sync_copy(z_hbm, acc.at[pl.ds(r0, ROWS_PER_TILE)])
    plsc.subcore_barrier()

    def issue_idx(k, m):
        off = base + k * CHUNK
        pltpu.async_copy(ei_hbm.at[0, pl.ds(off, CHUNK)], src[m], gi[m])
        pltpu.async_copy(ei_hbm.at[1, pl.ds(off, CHUNK)], dst[m], gi[m])

    def wait_idx(m):
        pltpu.make_async_copy(ei_hbm.at[0, pl.ds(0, CHUNK)], src[m], gi[m]).wait()
        pltpu.make_async_copy(ei_hbm.at[1, pl.ds(0, CHUNK)], dst[m], gi[m]).wait()

    def issue_gather(m):
        pltpu.async_copy(a_hbm.at[src[m]], ar[m], g[m])
        pltpu.async_copy(b_hbm.at[dst[m]], br[m], g[m])

    def wait_gather(m):
        pltpu.make_async_copy(a_hbm.at[src[m]], ar[m], g[m]).wait()
        pltpu.make_async_copy(b_hbm.at[dst[m]], br[m], g[m]).wait()

    def issue_scatter(m):
        pltpu.async_copy(mr[m], acc.at[dsts[m]], sc[m], add=True)

    def wait_scatter(m):
        pltpu.make_async_copy(mr[m], acc.at[dsts[m]], sc[m]).wait()

    def one_iter(k, gp, pp, pref_gather, pref_idx, wait_sc):
        wait_gather(gp)
        if wait_sc:
            wait_scatter(pp)
        for j in (0, LANES, CHUNK - LANES):
            dsts[pp][pl.ds(j, LANES)] = dst[gp][pl.ds(j, LANES)]
        if pref_idx:
            issue_idx(k + 3, gp)
        if pref_gather:
            m2 = (gp + 2) % 3
            wait_idx(m2)
            issue_gather(m2)

        @pl.loop(0, CHUNK)
        def _(i):
            elv = ar[gp][i, pl.ds(HF, LANES)]
            erv = br[gp][i, :]
            sv = elv + erv
            wv = jnp.exp(jnp.maximum(sv, sv * ALPHA))
            mr[pp][i, pl.ds(HF, LANES)] = wv
            for h in range(H):
                wb = _bcast_lane(wv, h)
                ftv = ar[gp][i, pl.ds(h * F, LANES)]
                mr[pp][i, pl.ds(h * F, LANES)] = ftv * wb

        issue_scatter(pp)

    issue_idx(0, 0)
    issue_idx(1, 1)
    issue_idx(2, 2)
    wait_idx(0)
    issue_gather(0)
    wait_idx(1)
    issue_gather(1)
    one_iter(0, 0, 0, True, True, False)
    one_iter(1, 1, 1, True, True, False)

    @pl.loop(0, (NK - 10) // 6)
    def _(j):
        k = 2 + j * 6
        for u in range(6):
            one_iter(k + u, (2 + u) % 3, u % 2, True, True, True)

    for k in range(NK - 8, NK):
        one_iter(k, k % 3, k % 2,
                 k + 2 <= NK - 1, k + 3 <= NK - 1, True)
    wait_scatter(0)
    wait_scatter(1)

    plsc.subcore_barrier()
    pltpu.sync_copy(acc.at[pl.ds(r0, ROWS_PER_TILE)],
                    out_hbm.at[c, pl.ds(r0, ROWS_PER_TILE)])


def _sc_edge(a, b, edge_index, zeros):
    mesh = plsc.VectorSubcoreMesh(core_axis_name="c", subcore_axis_name="s")
    kern = pl.kernel(
        _sc_edge_body,
        out_type=jax.ShapeDtypeStruct((2, N, AW), jnp.float32),
        mesh=mesh,
        scratch_types=(
            [pltpu.VMEM((CHUNK,), jnp.int32)] * 6
            + [pltpu.VMEM((CHUNK,), jnp.int32)] * 2
            + [pltpu.VMEM((CHUNK, AW), jnp.float32)] * 3
            + [pltpu.VMEM((CHUNK, BW), jnp.float32)] * 3
            + [pltpu.VMEM((CHUNK, AW), jnp.float32)] * 2
            + [pltpu.VMEM_SHARED((N, AW), jnp.float32)]
            + [pltpu.SemaphoreType.DMA] * 8
        ),
        compiler_params=pltpu.CompilerParams(use_tc_tiling_on_sc=False),
    )
    return kern(a, b, edge_index, zeros)


def _div_body(acc_ref, r_ref, o_ref):
    sacc = acc_ref[0] + acc_ref[1]
    den = sacc[:, HF:HF + H]
    dn = (((1,), (0,)), ((), ()))
    denb = lax.dot_general(den, r_ref[...], dn,
                           preferred_element_type=jnp.float32)
    num = sacc[:, :HF]
    o_ref[...] = jnp.where(denb > 0.0, num / denb, 0.0)


def _div(acc, r):
    return pl.pallas_call(
        _div_body,
        out_shape=jax.ShapeDtypeStruct((N, HF), jnp.float32),
    )(acc, r)


def kernel(feat, edge_index, W, attn_l, attn_r):
    f32 = jnp.float32
    wt = W.T.astype(f32)
    eye = jnp.eye(H, dtype=f32)
    wl = (attn_l[0][:, :, None] * eye[:, None, :]).reshape(HF, H)
    wr = (attn_r[0][:, :, None] * eye[:, None, :]).reshape(HF, H)
    wl = jnp.pad(wl, ((0, 0), (0, BW - H)))
    wr = jnp.pad(wr, ((0, 0), (0, BW - H)))
    ma = jnp.concatenate([wt, wt @ wl], axis=1)
    mb = wt @ wr

    a, b = _prep(feat, ma, mb)
    zeros = jnp.zeros((ROWS_PER_TILE, AW), f32)
    acc = _sc_edge(a, b, edge_index, zeros)

    r = jnp.repeat(eye, F, axis=1)
    out = _div(acc, r)
    return out.reshape(N, H, F)


# speedup vs baseline: 1.1742x; 1.0018x over previous
import functools

import jax
import jax.numpy as jnp
from jax import lax
from jax.experimental import pallas as pl
from jax.experimental.pallas import tpu as pltpu
from jax.experimental.pallas import tpu_sc as plsc

N = 10000
E = 320000
D = 128
H = 8
F = 16
HF = H * F
ALPHA = 0.2
AW = HF + 16
BW = 16
LANES = 16

CHUNK = 40
NTILES = 32
EDGES_PER_TILE = E // NTILES
NK = EDGES_PER_TILE // CHUNK
ROWS_PER_TILE = N // 16


def _prep_body(feat_ref, ma_ref, mb_ref, a_ref, b_ref):
    f = feat_ref[...]
    dn = (((1,), (0,)), ((), ()))
    a_ref[...] = lax.dot_general(f, ma_ref[...], dn,
                                 preferred_element_type=jnp.float32)
    b_ref[...] = lax.dot_general(f, mb_ref[...], dn,
                                 preferred_element_type=jnp.float32)


def _prep(feat, ma, mb):
    return pl.pallas_call(
        _prep_body,
        out_shape=(
            jax.ShapeDtypeStruct((N, AW), jnp.float32),
            jax.ShapeDtypeStruct((N, BW), jnp.float32),
        ),
    )(feat, ma, mb)


def _bcast_lane(vec, lane):
    idx = jnp.full((LANES, 1), lane, dtype=jnp.int32)
    dn = lax.GatherDimensionNumbers(
        offset_dims=(), collapsed_slice_dims=(0,), start_index_map=(0,))
    return lax.gather(vec, idx, dn, (1,),
                      mode=lax.GatherScatterMode.PROMISE_IN_BOUNDS)


def _sc_edge_body(a_hbm, b_hbm, ei_hbm, z_hbm, out_hbm,
                  src0, src1, src2, dst0, dst1, dst2, dsts0, dsts1,
                  ar0, ar1, ar2, br0, br1, br2, mr0, mr1, acc,
                  g0, g1, g2, gi0, gi1, gi2, sc0, sc1):
    c = lax.axis_index("c")
    s = lax.axis_index("s")
    wid = s * 2 + c
    base = wid * EDGES_PER_TILE

    src = (src0, src1, src2)
    dst = (dst0, dst1, dst2)
    dsts = (dsts0, dsts1)
    ar = (ar0, ar1, ar2)
    br = (br0, br1, br2)
    mr = (mr0, mr1)
    g = (g0, g1, g2)
    gi = (gi0, gi1, gi2)
    sc = (sc0, sc1)

    r0 = s * ROWS_PER_TILE

    def issue_idx(k, m):
        off = base + k * CHUNK
        pltpu.async_copy(ei_hbm.at[0, pl.ds(off, CHUNK)], src[m], gi[m])
        pltpu.async_copy(ei_hbm.at[1, pl.ds(off, CHUNK)], dst[m], gi[m])

    def wait_idx(m):
        pltpu.make_async_copy(ei_hbm.at[0, pl.ds(0, CHUNK)], src[m], gi[m]).wait()
        pltpu.make_async_copy(ei_hbm.at[1, pl.ds(0, CHUNK)], dst[m], gi[m]).wait()

    def issue_gather(m):
        pltpu.async_copy(a_hbm.at[src[m]], ar[m], g[m])
        pltpu.async_copy(b_hbm.at[dst[m]], br[m], g[m])

    def wait_gather(m):
        pltpu.make_async_copy(a_hbm.at[src[m]], ar[m], g[m]).wait()
        pltpu.make_async_copy(b_hbm.at[dst[m]], br[m], g[m]).wait()

    def issue_scatter(m):
        pltpu.async_copy(mr[m], acc.at[dsts[m]], sc[m], add=True)

    def wait_scatter(m):
        pltpu.make_async_copy(mr[m], acc.at[dsts[m]], sc[m]).wait()

    def one_iter(k, gp, pp, pref_gather, pref_idx, wait_sc):
        wait_gather(gp)
        if wait_sc:
            wait_scatter(pp)
        for j in (0, LANES, CHUNK - LANES):
            dsts[pp][pl.ds(j, LANES)] = dst[gp][pl.ds(j, LANES)]
        if pref_idx:
            issue_idx(k + 3, gp)
        if pref_gather:
            m2 = (gp + 2) % 3
            wait_idx(m2)
            issue_gather(m2)

        @pl.loop(0, CHUNK)
        def _(i):
            elv = ar[gp][i, pl.ds(HF, LANES)]
            erv = br[gp][i, :]
            sv = elv + erv
            wv = jnp.exp(jnp.maximum(sv, sv * ALPHA))
            mr[pp][i, pl.ds(HF, LANES)] = wv
            for h in range(H):
                wb = _bcast_lane(wv, h)
                ftv = ar[gp][i, pl.ds(h * F, LANES)]
                mr[pp][i, pl.ds(h * F, LANES)] = ftv * wb

        issue_scatter(pp)

    issue_idx(0, 0)
    issue_idx(1, 1)
    issue_idx(2, 2)
    pltpu.sync_copy(z_hbm, acc.at[pl.ds(r0, ROWS_PER_TILE)])
    wait_idx(0)
    issue_gather(0)
    wait_idx(1)
    issue_gather(1)
    plsc.subcore_barrier()
    one_iter(0, 0, 0, True, True, False)
    one_iter(1, 1, 1, True, True, False)

    @pl.loop(0, (NK - 10) // 6)
    def _(j):
        k = 2 + j * 6
        for u in range(6):
            one_iter(k + u, (2 + u) % 3, u % 2, True, True, True)

    for k in range(NK - 8, NK):
        one_iter(k, k % 3, k % 2,
                 k + 2 <= NK - 1, k + 3 <= NK - 1, True)
    wait_scatter(0)
    wait_scatter(1)

    plsc.subcore_barrier()
    pltpu.sync_copy(acc.at[pl.ds(r0, ROWS_PER_TILE)],
                    out_hbm.at[c, pl.ds(r0, ROWS_PER_TILE)])


def _sc_edge(a, b, edge_index, zeros):
    mesh = plsc.VectorSubcoreMesh(core_axis_name="c", subcore_axis_name="s")
    kern = pl.kernel(
        _sc_edge_body,
        out_type=jax.ShapeDtypeStruct((2, N, AW), jnp.float32),
        mesh=mesh,
        scratch_types=(
            [pltpu.VMEM((CHUNK,), jnp.int32)] * 6
            + [pltpu.VMEM((CHUNK,), jnp.int32)] * 2
            + [pltpu.VMEM((CHUNK, AW), jnp.float32)] * 3
            + [pltpu.VMEM((CHUNK, BW), jnp.float32)] * 3
            + [pltpu.VMEM((CHUNK, AW), jnp.float32)] * 2
            + [pltpu.VMEM_SHARED((N, AW), jnp.float32)]
            + [pltpu.SemaphoreType.DMA] * 8
        ),
        compiler_params=pltpu.CompilerParams(use_tc_tiling_on_sc=False),
    )
    return kern(a, b, edge_index, zeros)


def _div_body(acc_ref, r_ref, o_ref):
    sacc = acc_ref[0] + acc_ref[1]
    den = sacc[:, HF:HF + H]
    dn = (((1,), (0,)), ((), ()))
    denb = lax.dot_general(den, r_ref[...], dn,
                           preferred_element_type=jnp.float32)
    num = sacc[:, :HF]
    o_ref[...] = jnp.where(denb > 0.0, num / denb, 0.0)


def _div(acc, r):
    return pl.pallas_call(
        _div_body,
        out_shape=jax.ShapeDtypeStruct((N, HF), jnp.float32),
    )(acc, r)


def kernel(feat, edge_index, W, attn_l, attn_r):
    f32 = jnp.float32
    wt = W.T.astype(f32)
    eye = jnp.eye(H, dtype=f32)
    wl = (attn_l[0][:, :, None] * eye[:, None, :]).reshape(HF, H)
    wr = (attn_r[0][:, :, None] * eye[:, None, :]).reshape(HF, H)
    wl = jnp.pad(wl, ((0, 0), (0, BW - H)))
    wr = jnp.pad(wr, ((0, 0), (0, BW - H)))
    ma = jnp.concatenate([wt, wt @ wl], axis=1)
    mb = wt @ wr

    a, b = _prep(feat, ma, mb)
    zeros = jnp.zeros((ROWS_PER_TILE, AW), f32)
    acc = _sc_edge(a, b, edge_index, zeros)

    r = jnp.repeat(eye, F, axis=1)
    out = _div(acc, r)
    return out.reshape(N, H, F)
